# baseline (speedup 1.0000x reference)
import functools

import jax
import jax.numpy as jnp
from jax import lax
from jax.experimental import pallas as pl
from jax.experimental.pallas import tpu as pltpu
from jax.experimental.pallas import tpu_sc as plsc

_NB_CLASSES = 100000
_NB_REL = 1000
_DIM = 64
_B = 16384
_MARGIN = 0.01

_NC, _NS = 2, 16
_NW = _NC * _NS
_L = 16

_N_CLS = 13
_N_REL = 3

_WPAD = 128
_PER_W = _B // _NW
_SB = 32
_NSB = _PER_W // _SB
_NG = _SB // _L


def _vsqrt(x):
    i = plsc.bitcast(x, jnp.int32)
    i = jnp.int32(0x5F3759DF) - lax.shift_right_logical(i, 1)
    y = plsc.bitcast(i, jnp.float32)
    xh = 0.5 * x
    y = y * (1.5 - xh * y * y)
    y = y * (1.5 - xh * y * y)
    y = y * (1.5 - xh * y * y)
    return x * y


def _relu(v):
    return jnp.maximum(v, 0.0)


def _reg(acc):
    return jnp.abs(_vsqrt(acc) - 1.0)


def _sc_loss_kernel(cls_hbm, cidx_hbm, rel_hbm, ridx_hbm, out_hbm,
                    cidx_v, ridx_v, *rest):
    cbufs = rest[:_N_CLS]
    rbufs = rest[_N_CLS:_N_CLS + _N_REL]
    out_v = rest[_N_CLS + _N_REL]
    sem = rest[_N_CLS + _N_REL + 1]

    wid = lax.axis_index("s") * _NC + lax.axis_index("c")

    pltpu.sync_copy(cidx_hbm.at[pl.ds(wid * (_N_CLS * _NSB), _N_CLS * _NSB)], cidx_v)
    pltpu.sync_copy(ridx_hbm.at[pl.ds(wid * (_N_REL * _NSB), _N_REL * _NSB)], ridx_v)

    iota = lax.iota(jnp.int32, _L)
    col_rad = jnp.full((_L,), _DIM, jnp.int32)
    zero = jnp.zeros((_L,), jnp.float32)

    def gcol(buf, rows, col):
        return plsc.load_gather(buf, (rows, col))

    def rad(buf, rows):
        return jnp.abs(gcol(buf, rows, col_rad))

    def sb_body(s, _):
        handles = [
            pltpu.async_copy(cls_hbm.at[cidx_v.at[r * _NSB + s]], cbufs[r], sem)
            for r in range(_N_CLS)
        ] + [
            pltpu.async_copy(rel_hbm.at[ridx_v.at[q * _NSB + s]], rbufs[q], sem)
            for q in range(_N_REL)
        ]
        for h in handles:
            h.wait()

        def g_body(g, _):
            rows = iota + g * _L

            def pair_term(ba, bb):
                def body(dd, accs):
                    e, a, b = accs
                    col = jnp.full((_L,), dd, jnp.int32)
                    va = gcol(ba, rows, col)
                    vb = gcol(bb, rows, col)
                    df = va - vb
                    return (e + df * df, a + va * va, b + vb * vb)
                return lax.fori_loop(0, _DIM, body, (zero, zero, zero),
                                     unroll=4)

            def rel_term(ba, bb, br, sign):
                def body(dd, accs):
                    e, a, b = accs
                    col = jnp.full((_L,), dd, jnp.int32)
                    va = gcol(ba, rows, col)
                    vb = gcol(bb, rows, col)
                    vr = gcol(br, rows, col)
                    df = va + sign * vr - vb
                    return (e + df * df, a + va * va, b + vb * vb)
                return lax.fori_loop(0, _DIM, body, (zero, zero, zero),
                                     unroll=4)

            e, a, b = pair_term(cbufs[0], cbufs[1])
            rc, rd = rad(cbufs[0], rows), rad(cbufs[1], rows)
            total = (_relu(_vsqrt(e) + rc - rd - _MARGIN)
                     + _reg(a) + _reg(b))

            def nf2_term(ba, bb, bc):
                def body(dd, accs):
                    e21, e22, e23, a_, b_, c_ = accs
                    col = jnp.full((_L,), dd, jnp.int32)
                    va = gcol(ba, rows, col)
                    vb = gcol(bb, rows, col)
                    vc = gcol(bc, rows, col)
                    d1 = vb - va
                    d2 = vc - va
                    d3 = vc - vb
                    return (e21 + d1 * d1, e22 + d2 * d2, e23 + d3 * d3,
                            a_ + va * va, b_ + vb * vb, c_ + vc * vc)
                return lax.fori_loop(0, _DIM, body, (zero,) * 6, unroll=4)

            e21, e22, e23, a, b, c = nf2_term(cbufs[2], cbufs[3], cbufs[4])
            rc, rd = rad(cbufs[2], rows), rad(cbufs[3], rows)
            total += (_relu(_vsqrt(e21) - (rc + rd) - _MARGIN)
                      + _relu(_vsqrt(e22) - rc - _MARGIN)
                      + _relu(_vsqrt(e23) - rd - _MARGIN)
                      + _reg(a) + _reg(b) + _reg(c))

            e, a, b = rel_term(cbufs[5], cbufs[6], rbufs[0], 1.0)
            rc, rd = rad(cbufs[5], rows), rad(cbufs[6], rows)
            total += (_relu(_vsqrt(e) + rc - rd - _MARGIN)
                      + _reg(a) + _reg(b))

            e, a, b = rel_term(cbufs[7], cbufs[8], rbufs[1], -1.0)
            rc, rd = rad(cbufs[7], rows), rad(cbufs[8], rows)
            total += (_relu(_vsqrt(e) - (rc + rd) - _MARGIN)
                      + _reg(a) + _reg(b))

            e, a, b = pair_term(cbufs[9], cbufs[10])
            rc, rd = rad(cbufs[9], rows), rad(cbufs[10], rows)
            total += (_relu(rc + rd - _vsqrt(e) + _MARGIN)
                      + _reg(a) + _reg(b))

            e, a, b = rel_term(cbufs[11], cbufs[12], rbufs[2], 1.0)
            rc, rd = rad(cbufs[11], rows), rad(cbufs[12], rows)
            total += (_relu(rc + rd + _MARGIN - _vsqrt(e))
                      + _reg(a) + _reg(b))

            out_v[pl.ds(s * _SB + g * _L, _L)] = total
            return ()

        lax.fori_loop(0, _NG, g_body, (), unroll=False)
        return ()

    lax.fori_loop(0, _NSB, sb_body, (), unroll=False)

    pltpu.sync_copy(out_v, out_hbm.at[pl.ds(wid * _PER_W, _PER_W)])


@functools.lru_cache(maxsize=1)
def _sc_loss():
    return pl.kernel(
        _sc_loss_kernel,
        out_type=jax.ShapeDtypeStruct((_B,), jnp.float32),
        mesh=plsc.VectorSubcoreMesh(core_axis_name="c", subcore_axis_name="s"),
        compiler_params=pltpu.CompilerParams(use_tc_tiling_on_sc=False,
                                             needs_layout_passes=False),
        scratch_types=(
            [pltpu.VMEM((_N_CLS * _NSB, _SB), jnp.int32),
             pltpu.VMEM((_N_REL * _NSB, _SB), jnp.int32)]
            + [pltpu.VMEM((_SB, _WPAD), jnp.float32)] * _N_CLS
            + [pltpu.VMEM((_SB, _DIM), jnp.float32)] * _N_REL
            + [pltpu.VMEM((_PER_W,), jnp.float32),
               pltpu.SemaphoreType.DMA]
        ),
    )


def kernel(nf1, nf2, nf3, nf4, dis, top, nf3_neg, cls_emb, rel_emb):
    del top
    i32 = jnp.int32
    cidx = jnp.stack([
        nf1[:, 0], nf1[:, 1],
        nf2[:, 0], nf2[:, 1], nf2[:, 2],
        nf3[:, 0], nf3[:, 2],
        nf4[:, 1], nf4[:, 2],
        dis[:, 0], dis[:, 1],
        nf3_neg[:, 0], nf3_neg[:, 2],
    ]).astype(i32)
    ridx = jnp.stack([nf3[:, 1], nf4[:, 0], nf3_neg[:, 1]]).astype(i32)
    cidx = (cidx.reshape(_N_CLS, _NW, _NSB, _SB)
            .transpose(1, 0, 2, 3).reshape(_NW * _N_CLS * _NSB, _SB))
    ridx = (ridx.reshape(_N_REL, _NW, _NSB, _SB)
            .transpose(1, 0, 2, 3).reshape(_NW * _N_REL * _NSB, _SB))

    cls_pad = jnp.pad(cls_emb, ((0, 0), (0, _WPAD - (_DIM + 1))))
    out = _sc_loss()(cls_pad, cidx, rel_emb, ridx)
    return out.reshape(_B, 1)


# speedup vs baseline: 1.4860x; 1.4860x over previous
import functools

import jax
import jax.numpy as jnp
from jax import lax
from jax.experimental import pallas as pl
from jax.experimental.pallas import tpu as pltpu
from jax.experimental.pallas import tpu_sc as plsc

_NB_CLASSES = 100000
_NB_REL = 1000
_DIM = 64
_B = 16384
_MARGIN = 0.01

_NC, _NS = 2, 16
_NW = _NC * _NS
_L = 16

_N_CLS = 13
_N_REL = 3

_WPAD = 72
_PER_W = _B // _NW
_SB = 32
_NSB = _PER_W // _SB
_NG = _SB // _L


def _vsqrt(x):
    i = plsc.bitcast(x, jnp.int32)
    i = jnp.int32(0x5F3759DF) - lax.shift_right_logical(i, 1)
    y = plsc.bitcast(i, jnp.float32)
    xh = 0.5 * x
    y = y * (1.5 - xh * y * y)
    y = y * (1.5 - xh * y * y)
    y = y * (1.5 - xh * y * y)
    return x * y


def _relu(v):
    return jnp.maximum(v, 0.0)


def _reg(acc):
    return jnp.abs(_vsqrt(acc) - 1.0)


def _sc_loss_kernel(cls_hbm, cidx_hbm, rel_hbm, ridx_hbm, out_hbm,
                    cidx_v, ridx_v, *rest):
    nbuf = _N_CLS + _N_REL
    sets = (rest[:nbuf], rest[nbuf:2 * nbuf])
    out_v = rest[2 * nbuf]
    sems = (rest[2 * nbuf + 1], rest[2 * nbuf + 2])

    wid = lax.axis_index("s") * _NC + lax.axis_index("c")

    pltpu.sync_copy(cidx_hbm.at[pl.ds(wid * (_N_CLS * _NSB), _N_CLS * _NSB)], cidx_v)
    pltpu.sync_copy(ridx_hbm.at[pl.ds(wid * (_N_REL * _NSB), _N_REL * _NSB)], ridx_v)

    iota = lax.iota(jnp.int32, _L)
    col_rad = jnp.full((_L,), _DIM, jnp.int32)
    zero = jnp.zeros((_L,), jnp.float32)

    def gcol(buf, rows, col):
        return plsc.load_gather(buf, (rows, col))

    def rad(buf, rows):
        return jnp.abs(gcol(buf, rows, col_rad))

    def fire(p, s):
        bufs, sem = sets[p], sems[p]
        for r in range(_N_CLS):
            pltpu.async_copy(cls_hbm.at[cidx_v.at[r * _NSB + s]], bufs[r], sem)
        for q in range(_N_REL):
            pltpu.async_copy(rel_hbm.at[ridx_v.at[q * _NSB + s]], bufs[_N_CLS + q], sem)

    def drain(p):
        bufs, sem = sets[p], sems[p]
        for r in range(_N_CLS):
            pltpu.make_async_copy(cls_hbm.at[pl.ds(0, _SB)], bufs[r], sem).wait()
        for q in range(_N_REL):
            pltpu.make_async_copy(rel_hbm.at[pl.ds(0, _SB)], bufs[_N_CLS + q], sem).wait()

    def compute(p, s):
        bufs = sets[p]

        def g_body(g, _):
            rows = iota + g * _L

            def pair_term(ba, bb):
                def body(dd, accs):
                    e, a, b = accs
                    col = jnp.full((_L,), dd, jnp.int32)
                    va = gcol(ba, rows, col)
                    vb = gcol(bb, rows, col)
                    df = va - vb
                    return (e + df * df, a + va * va, b + vb * vb)
                return lax.fori_loop(0, _DIM, body, (zero, zero, zero),
                                     unroll=4)

            def rel_term(ba, bb, br, sign):
                def body(dd, accs):
                    e, a, b = accs
                    col = jnp.full((_L,), dd, jnp.int32)
                    va = gcol(ba, rows, col)
                    vb = gcol(bb, rows, col)
                    vr = gcol(br, rows, col)
                    df = va + sign * vr - vb
                    return (e + df * df, a + va * va, b + vb * vb)
                return lax.fori_loop(0, _DIM, body, (zero, zero, zero),
                                     unroll=4)

            e, a, b = pair_term(bufs[0], bufs[1])
            rc, rd = rad(bufs[0], rows), rad(bufs[1], rows)
            total = (_relu(_vsqrt(e) + rc - rd - _MARGIN)
                     + _reg(a) + _reg(b))

            def nf2_term(ba, bb, bc):
                def body(dd, accs):
                    e21, e22, e23, a_, b_, c_ = accs
                    col = jnp.full((_L,), dd, jnp.int32)
                    va = gcol(ba, rows, col)
                    vb = gcol(bb, rows, col)
                    vc = gcol(bc, rows, col)
                    d1 = vb - va
                    d2 = vc - va
                    d3 = vc - vb
                    return (e21 + d1 * d1, e22 + d2 * d2, e23 + d3 * d3,
                            a_ + va * va, b_ + vb * vb, c_ + vc * vc)
                return lax.fori_loop(0, _DIM, body, (zero,) * 6, unroll=4)

            e21, e22, e23, a, b, c = nf2_term(bufs[2], bufs[3], bufs[4])
            rc, rd = rad(bufs[2], rows), rad(bufs[3], rows)
            total += (_relu(_vsqrt(e21) - (rc + rd) - _MARGIN)
                      + _relu(_vsqrt(e22) - rc - _MARGIN)
                      + _relu(_vsqrt(e23) - rd - _MARGIN)
                      + _reg(a) + _reg(b) + _reg(c))

            e, a, b = rel_term(bufs[5], bufs[6], bufs[_N_CLS + 0], 1.0)
            rc, rd = rad(bufs[5], rows), rad(bufs[6], rows)
            total += (_relu(_vsqrt(e) + rc - rd - _MARGIN)
                      + _reg(a) + _reg(b))

            e, a, b = rel_term(bufs[7], bufs[8], bufs[_N_CLS + 1], -1.0)
            rc, rd = rad(bufs[7], rows), rad(bufs[8], rows)
            total += (_relu(_vsqrt(e) - (rc + rd) - _MARGIN)
                      + _reg(a) + _reg(b))

            e, a, b = pair_term(bufs[9], bufs[10])
            rc, rd = rad(bufs[9], rows), rad(bufs[10], rows)
            total += (_relu(rc + rd - _vsqrt(e) + _MARGIN)
                      + _reg(a) + _reg(b))

            e, a, b = rel_term(bufs[11], bufs[12], bufs[_N_CLS + 2], 1.0)
            rc, rd = rad(bufs[11], rows), rad(bufs[12], rows)
            total += (_relu(rc + rd + _MARGIN - _vsqrt(e))
                      + _reg(a) + _reg(b))

            out_v[pl.ds(s * _SB + g * _L, _L)] = total
            return ()

        lax.fori_loop(0, _NG, g_body, (), unroll=False)

    fire(0, 0)

    def body(i, _):
        s0 = 2 * i
        s1 = s0 + 1
        fire(1, s1)
        drain(0)
        compute(0, s0)

        @pl.when(i + 1 < _NSB // 2)
        def _():
            fire(0, s0 + 2)

        drain(1)
        compute(1, s1)
        return ()

    lax.fori_loop(0, _NSB // 2, body, (), unroll=False)

    pltpu.sync_copy(out_v, out_hbm.at[pl.ds(wid * _PER_W, _PER_W)])


@functools.lru_cache(maxsize=1)
def _sc_loss():
    role_bufs = ([pltpu.VMEM((_SB, _WPAD), jnp.float32)] * _N_CLS
                 + [pltpu.VMEM((_SB, _DIM), jnp.float32)] * _N_REL)
    return pl.kernel(
        _sc_loss_kernel,
        out_type=jax.ShapeDtypeStruct((_B,), jnp.float32),
        mesh=plsc.VectorSubcoreMesh(core_axis_name="c", subcore_axis_name="s"),
        compiler_params=pltpu.CompilerParams(use_tc_tiling_on_sc=False,
                                             needs_layout_passes=False),
        scratch_types=(
            [pltpu.VMEM((_N_CLS * _NSB, _SB), jnp.int32),
             pltpu.VMEM((_N_REL * _NSB, _SB), jnp.int32)]
            + role_bufs + role_bufs
            + [pltpu.VMEM((_PER_W,), jnp.float32),
               pltpu.SemaphoreType.DMA,
               pltpu.SemaphoreType.DMA]
        ),
    )


def kernel(nf1, nf2, nf3, nf4, dis, top, nf3_neg, cls_emb, rel_emb):
    del top
    i32 = jnp.int32
    cidx = jnp.stack([
        nf1[:, 0], nf1[:, 1],
        nf2[:, 0], nf2[:, 1], nf2[:, 2],
        nf3[:, 0], nf3[:, 2],
        nf4[:, 1], nf4[:, 2],
        dis[:, 0], dis[:, 1],
        nf3_neg[:, 0], nf3_neg[:, 2],
    ]).astype(i32)
    ridx = jnp.stack([nf3[:, 1], nf4[:, 0], nf3_neg[:, 1]]).astype(i32)
    cidx = (cidx.reshape(_N_CLS, _NW, _NSB, _SB)
            .transpose(1, 0, 2, 3).reshape(_NW * _N_CLS * _NSB, _SB))
    ridx = (ridx.reshape(_N_REL, _NW, _NSB, _SB)
            .transpose(1, 0, 2, 3).reshape(_NW * _N_REL * _NSB, _SB))

    cls_pad = jnp.pad(cls_emb, ((0, 0), (0, _WPAD - (_DIM + 1))))
    out = _sc_loss()(cls_pad, cidx, rel_emb, ridx)
    return out.reshape(_B, 1)


# speedup vs baseline: 1.5254x; 1.0266x over previous
import functools

import jax
import jax.numpy as jnp
from jax import lax
from jax.experimental import pallas as pl
from jax.experimental.pallas import tpu as pltpu
from jax.experimental.pallas import tpu_sc as plsc

_NB_CLASSES = 100000
_NB_REL = 1000
_DIM = 64
_B = 16384
_MARGIN = 0.01

_NC, _NS = 2, 16
_NW = _NC * _NS
_L = 16

_N_CLS = 13
_N_REL = 3

_WPAD = 72
_PER_W = _B // _NW
_SB = 32
_NSB = _PER_W // _SB
_NG = _SB // _L


def _vsqrt(x):
    i = plsc.bitcast(x, jnp.int32)
    i = jnp.int32(0x5F3759DF) - lax.shift_right_logical(i, 1)
    y = plsc.bitcast(i, jnp.float32)
    xh = 0.5 * x
    y = y * (1.5 - xh * y * y)
    y = y * (1.5 - xh * y * y)
    y = y * (1.5 - xh * y * y)
    return x * y


def _relu(v):
    return jnp.maximum(v, 0.0)


def _reg(acc):
    return jnp.abs(_vsqrt(acc) - 1.0)


def _sc_loss_kernel(cls_hbm, cidx_hbm, rel_hbm, ridx_hbm, out_hbm,
                    cidx_v, ridx_v, *rest):
    nbuf = _N_CLS + _N_REL
    sets = (rest[:nbuf], rest[nbuf:2 * nbuf])
    out_v = rest[2 * nbuf]
    sems = (rest[2 * nbuf + 1], rest[2 * nbuf + 2])

    wid = lax.axis_index("s") * _NC + lax.axis_index("c")

    pltpu.sync_copy(cidx_hbm.at[pl.ds(wid * (_N_CLS * _NSB), _N_CLS * _NSB)], cidx_v)
    pltpu.sync_copy(ridx_hbm.at[pl.ds(wid * (_N_REL * _NSB), _N_REL * _NSB)], ridx_v)

    iota = lax.iota(jnp.int32, _L)
    col_rad = jnp.full((_L,), _DIM, jnp.int32)
    zero = jnp.zeros((_L,), jnp.float32)

    def gcol(buf, rows, col):
        return plsc.load_gather(buf, (rows, col))

    def rad(buf, rows):
        return jnp.abs(gcol(buf, rows, col_rad))

    def fire(p, s):
        bufs, sem = sets[p], sems[p]
        for r in range(_N_CLS):
            pltpu.async_copy(cls_hbm.at[cidx_v.at[r * _NSB + s]], bufs[r], sem)
        for q in range(_N_REL):
            pltpu.async_copy(rel_hbm.at[ridx_v.at[q * _NSB + s]], bufs[_N_CLS + q], sem)

    def drain(p):
        bufs, sem = sets[p], sems[p]
        for r in range(_N_CLS):
            pltpu.make_async_copy(cls_hbm.at[pl.ds(0, _SB)], bufs[r], sem).wait()
        for q in range(_N_REL):
            pltpu.make_async_copy(rel_hbm.at[pl.ds(0, _SB)], bufs[_N_CLS + q], sem).wait()

    def compute(p, s):
        bufs = sets[p]

        def g_body(g, _):
            rows = iota + g * _L

            def pair_term(ba, bb):
                def body(dd, accs):
                    e, a, b = accs
                    col = jnp.full((_L,), dd, jnp.int32)
                    va = gcol(ba, rows, col)
                    vb = gcol(bb, rows, col)
                    df = va - vb
                    return (e + df * df, a + va * va, b + vb * vb)
                return lax.fori_loop(0, _DIM, body, (zero, zero, zero),
                                     unroll=8)

            def rel_term(ba, bb, br, sign):
                def body(dd, accs):
                    e, a, b = accs
                    col = jnp.full((_L,), dd, jnp.int32)
                    va = gcol(ba, rows, col)
                    vb = gcol(bb, rows, col)
                    vr = gcol(br, rows, col)
                    df = va + sign * vr - vb
                    return (e + df * df, a + va * va, b + vb * vb)
                return lax.fori_loop(0, _DIM, body, (zero, zero, zero),
                                     unroll=8)

            e, a, b = pair_term(bufs[0], bufs[1])
            rc, rd = rad(bufs[0], rows), rad(bufs[1], rows)
            total = (_relu(_vsqrt(e) + rc - rd - _MARGIN)
                     + _reg(a) + _reg(b))

            def nf2_term(ba, bb, bc):
                def body(dd, accs):
                    e21, e22, e23, a_, b_, c_ = accs
                    col = jnp.full((_L,), dd, jnp.int32)
                    va = gcol(ba, rows, col)
                    vb = gcol(bb, rows, col)
                    vc = gcol(bc, rows, col)
                    d1 = vb - va
                    d2 = vc - va
                    d3 = vc - vb
                    return (e21 + d1 * d1, e22 + d2 * d2, e23 + d3 * d3,
                            a_ + va * va, b_ + vb * vb, c_ + vc * vc)
                return lax.fori_loop(0, _DIM, body, (zero,) * 6, unroll=8)

            e21, e22, e23, a, b, c = nf2_term(bufs[2], bufs[3], bufs[4])
            rc, rd = rad(bufs[2], rows), rad(bufs[3], rows)
            total += (_relu(_vsqrt(e21) - (rc + rd) - _MARGIN)
                      + _relu(_vsqrt(e22) - rc - _MARGIN)
                      + _relu(_vsqrt(e23) - rd - _MARGIN)
                      + _reg(a) + _reg(b) + _reg(c))

            e, a, b = rel_term(bufs[5], bufs[6], bufs[_N_CLS + 0], 1.0)
            rc, rd = rad(bufs[5], rows), rad(bufs[6], rows)
            total += (_relu(_vsqrt(e) + rc - rd - _MARGIN)
                      + _reg(a) + _reg(b))

            e, a, b = rel_term(bufs[7], bufs[8], bufs[_N_CLS + 1], -1.0)
            rc, rd = rad(bufs[7], rows), rad(bufs[8], rows)
            total += (_relu(_vsqrt(e) - (rc + rd) - _MARGIN)
                      + _reg(a) + _reg(b))

            e, a, b = pair_term(bufs[9], bufs[10])
            rc, rd = rad(bufs[9], rows), rad(bufs[10], rows)
            total += (_relu(rc + rd - _vsqrt(e) + _MARGIN)
                      + _reg(a) + _reg(b))

            e, a, b = rel_term(bufs[11], bufs[12], bufs[_N_CLS + 2], 1.0)
            rc, rd = rad(bufs[11], rows), rad(bufs[12], rows)
            total += (_relu(rc + rd + _MARGIN - _vsqrt(e))
                      + _reg(a) + _reg(b))

            out_v[pl.ds(s * _SB + g * _L, _L)] = total
            return ()

        lax.fori_loop(0, _NG, g_body, (), unroll=False)

    fire(0, 0)

    def body(i, _):
        s0 = 2 * i
        s1 = s0 + 1
        fire(1, s1)
        drain(0)
        compute(0, s0)

        @pl.when(i + 1 < _NSB // 2)
        def _():
            fire(0, s0 + 2)

        drain(1)
        compute(1, s1)
        return ()

    lax.fori_loop(0, _NSB // 2, body, (), unroll=False)

    pltpu.sync_copy(out_v, out_hbm.at[pl.ds(wid * _PER_W, _PER_W)])


@functools.lru_cache(maxsize=1)
def _sc_loss():
    role_bufs = ([pltpu.VMEM((_SB, _WPAD), jnp.float32)] * _N_CLS
                 + [pltpu.VMEM((_SB, _DIM), jnp.float32)] * _N_REL)
    return pl.kernel(
        _sc_loss_kernel,
        out_type=jax.ShapeDtypeStruct((_B,), jnp.float32),
        mesh=plsc.VectorSubcoreMesh(core_axis_name="c", subcore_axis_name="s"),
        compiler_params=pltpu.CompilerParams(use_tc_tiling_on_sc=False,
                                             needs_layout_passes=False),
        scratch_types=(
            [pltpu.VMEM((_N_CLS * _NSB, _SB), jnp.int32),
             pltpu.VMEM((_N_REL * _NSB, _SB), jnp.int32)]
            + role_bufs + role_bufs
            + [pltpu.VMEM((_PER_W,), jnp.float32),
               pltpu.SemaphoreType.DMA,
               pltpu.SemaphoreType.DMA]
        ),
    )


def kernel(nf1, nf2, nf3, nf4, dis, top, nf3_neg, cls_emb, rel_emb):
    del top
    i32 = jnp.int32
    cidx = jnp.stack([
        nf1[:, 0], nf1[:, 1],
        nf2[:, 0], nf2[:, 1], nf2[:, 2],
        nf3[:, 0], nf3[:, 2],
        nf4[:, 1], nf4[:, 2],
        dis[:, 0], dis[:, 1],
        nf3_neg[:, 0], nf3_neg[:, 2],
    ]).astype(i32)
    ridx = jnp.stack([nf3[:, 1], nf4[:, 0], nf3_neg[:, 1]]).astype(i32)
    cidx = (cidx.reshape(_N_CLS, _NW, _NSB, _SB)
            .transpose(1, 0, 2, 3).reshape(_NW * _N_CLS * _NSB, _SB))
    ridx = (ridx.reshape(_N_REL, _NW, _NSB, _SB)
            .transpose(1, 0, 2, 3).reshape(_NW * _N_REL * _NSB, _SB))

    cls_pad = jnp.pad(cls_emb, ((0, 0), (0, _WPAD - (_DIM + 1))))
    out = _sc_loss()(cls_pad, cidx, rel_emb, ridx)
    return out.reshape(_B, 1)
